# baseline (speedup 1.0000x reference)
import functools

import jax
import jax.numpy as jnp
from jax import lax
from jax.experimental import pallas as pl
from jax.experimental.pallas import tpu as pltpu
from jax.experimental.pallas import tpu_sc as plsc

NUM_ENTITY = 10000
NUM_RELATION = 200
NUM_FACT = 320000
DIM = 128

NC = 2
NS = 16
NW = NC * NS
L = 16

FACTS_PER_W = NUM_FACT // NW
BLK = 80
CHUNK = 2000
BLKS_PER_CHUNK = CHUNK // BLK
PAIRS = (BLKS_PER_CHUNK - 1) // 2
CHUNKS = FACTS_PER_W // CHUNK
ROWS_PER_TILE = 624
REM_ROWS = NUM_ENTITY - NS * ROWS_PER_TILE

_mesh = plsc.VectorSubcoreMesh(
    core_axis_name="c", subcore_axis_name="s", num_cores=NC, num_subcores=NS)


@functools.partial(
    pl.kernel,
    out_type=jax.ShapeDtypeStruct((NC * NUM_ENTITY, DIM), jnp.float32),
    mesh=_mesh,
    scratch_types=dict(
        hd_st=pltpu.VMEM((CHUNK,), jnp.int32),
        rl_st=pltpu.VMEM((CHUNK,), jnp.int32),
        bi_st=pltpu.VMEM((CHUNK,), jnp.int32),
        tl_st=pltpu.VMEM((CHUNK,), jnp.int32),
        ridx0=pltpu.VMEM((BLK,), jnp.int32),
        ridx1=pltpu.VMEM((BLK,), jnp.int32),
        tidx0=pltpu.VMEM((BLK,), jnp.int32),
        tidx1=pltpu.VMEM((BLK,), jnp.int32),
        hbuf0=pltpu.VMEM((BLK, DIM // 2), jnp.int32),
        hbuf1=pltpu.VMEM((BLK, DIM // 2), jnp.int32),
        rbuf0=pltpu.VMEM((BLK, DIM // 2), jnp.int32),
        rbuf1=pltpu.VMEM((BLK, DIM // 2), jnp.int32),
        pbuf0=pltpu.VMEM((BLK, DIM), jnp.float32),
        pbuf1=pltpu.VMEM((BLK, DIM), jnp.float32),
        accum=pltpu.VMEM_SHARED((NUM_ENTITY, DIM), jnp.float32),
        sem_st=pltpu.SemaphoreType.DMA,
        sem_h0=pltpu.SemaphoreType.DMA,
        sem_h1=pltpu.SemaphoreType.DMA,
        sem_r0=pltpu.SemaphoreType.DMA,
        sem_r1=pltpu.SemaphoreType.DMA,
    ),
    compiler_params=pltpu.CompilerParams(use_tc_tiling_on_sc=False),
)
def _sc_message_pass(entity_hbm, rel_hbm, heads_hbm, rels_hbm, ids_hbm,
                     tails_hbm, part_hbm, hd_st, rl_st, bi_st, tl_st, ridx0,
                     ridx1, tidx0, tidx1, hbuf0, hbuf1, rbuf0, rbuf1, pbuf0,
                     pbuf1, accum, sem_st, sem_h0, sem_h1, sem_r0, sem_r1):
  core = lax.axis_index("c")
  sid = lax.axis_index("s")
  w = core * NS + sid

  zero = jnp.zeros((L,), jnp.float32)

  def _zrow(r, _):
    for j in range(DIM // L):
      pbuf0[r, pl.ds(j * L, L)] = zero
    return 0
  lax.fori_loop(0, BLK, _zrow, 0)
  for k in range(7):
    pltpu.sync_copy(pbuf0,
                    accum.at[pl.ds(sid * ROWS_PER_TILE + k * BLK, BLK)])
  pltpu.sync_copy(pbuf0.at[pl.ds(0, 64)],
                  accum.at[pl.ds(sid * ROWS_PER_TILE + 7 * BLK, 64)])
  @pl.when(sid == NS - 1)
  def _zero_tail():
    pltpu.sync_copy(pbuf0.at[pl.ds(0, REM_ROWS)],
                    accum.at[pl.ds(NS * ROWS_PER_TILE, REM_ROWS)])
  plsc.subcore_barrier()

  bufs = (
      (ridx0, tidx0, hbuf0, rbuf0, sem_h0, sem_r0, pbuf0),
      (ridx1, tidx1, hbuf1, rbuf1, sem_h1, sem_r1, pbuf1),
  )

  def _idx(off, p):
    ridx, tidx = bufs[p][0], bufs[p][1]
    for j in range(BLK // L):
      s = pl.ds(j * L, L)
      src = pl.ds(off + j * L, L)
      ridx[s] = rl_st[src] + bi_st[src] * NUM_RELATION
      tidx[s] = tl_st[src]

  def _issue_gathers(off, p):
    ridx, _, hbuf, rbuf, sem_h, sem_r = bufs[p][:6]
    pltpu.async_copy(entity_hbm.at[hd_st.at[pl.ds(off, BLK)]], hbuf, sem_h)
    pltpu.async_copy(rel_hbm.at[ridx], rbuf, sem_r)

  def _wait_gathers(off, p):
    ridx, _, hbuf, rbuf, sem_h, sem_r = bufs[p][:6]
    pltpu.make_async_copy(entity_hbm.at[hd_st.at[pl.ds(off, BLK)]], hbuf,
                          sem_h).wait()
    pltpu.make_async_copy(rel_hbm.at[ridx], rbuf, sem_r).wait()

  hi_mask = jnp.full((L,), -65536, jnp.int32)
  sixteen = jnp.full((L,), 16, jnp.int32)

  def _mul(p):
    hbuf, rbuf, pbuf = bufs[p][2], bufs[p][3], bufs[p][6]
    def _mrow(r, _):
      for j in range(DIM // (2 * L)):
        s = pl.ds(j * L, L)
        wh = hbuf[r, s]
        wr = rbuf[r, s]
        lo = (lax.bitcast_convert_type(lax.shift_left(wh, sixteen), jnp.float32)
              * lax.bitcast_convert_type(lax.shift_left(wr, sixteen), jnp.float32))
        hi = (lax.bitcast_convert_type(wh & hi_mask, jnp.float32)
              * lax.bitcast_convert_type(wr & hi_mask, jnp.float32))
        pbuf[r, pl.ds(2 * j * L, L)] = lo
        pbuf[r, pl.ds((2 * j + 1) * L, L)] = hi
      return 0
    lax.fori_loop(0, BLK, _mrow, 0)

  def _scatter(p):
    tidx, pbuf = bufs[p][1], bufs[p][6]
    pltpu.sync_copy(pbuf, accum.at[tidx], add=True)

  def _chunk(c, _):
    base = w * FACTS_PER_W + c * CHUNK
    cps = [
        pltpu.async_copy(heads_hbm.at[pl.ds(base, CHUNK)], hd_st, sem_st),
        pltpu.async_copy(rels_hbm.at[pl.ds(base, CHUNK)], rl_st, sem_st),
        pltpu.async_copy(ids_hbm.at[pl.ds(base, CHUNK)], bi_st, sem_st),
        pltpu.async_copy(tails_hbm.at[pl.ds(base, CHUNK)], tl_st, sem_st),
    ]
    for cp in cps:
      cp.wait()

    _idx(0, 0)
    _issue_gathers(0, 0)

    def _pair(i, _):
      b1 = 2 * i + 1
      _idx(b1 * BLK, 1)
      _issue_gathers(b1 * BLK, 1)
      _wait_gathers((b1 - 1) * BLK, 0)
      _mul(0)
      _scatter(0)

      b2 = 2 * i + 2
      _idx(b2 * BLK, 0)
      _issue_gathers(b2 * BLK, 0)
      _wait_gathers((b2 - 1) * BLK, 1)
      _mul(1)
      _scatter(1)
      return 0

    lax.fori_loop(0, PAIRS, _pair, 0)

    _wait_gathers((BLKS_PER_CHUNK - 1) * BLK, 0)
    _mul(0)
    _scatter(0)
    return 0

  lax.fori_loop(0, CHUNKS, _chunk, 0)

  plsc.subcore_barrier()
  for k in range(7):
    r0 = sid * ROWS_PER_TILE + k * BLK
    pltpu.sync_copy(accum.at[pl.ds(r0, BLK)], pbuf0)
    pltpu.sync_copy(pbuf0, part_hbm.at[pl.ds(core * NUM_ENTITY + r0, BLK)])
  r0 = sid * ROWS_PER_TILE + 7 * BLK
  pltpu.sync_copy(accum.at[pl.ds(r0, 64)], pbuf0.at[pl.ds(0, 64)])
  pltpu.sync_copy(pbuf0.at[pl.ds(0, 64)],
                  part_hbm.at[pl.ds(core * NUM_ENTITY + r0, 64)])
  @pl.when(sid == NS - 1)
  def _drain_tail():
    r1 = NS * ROWS_PER_TILE
    pltpu.sync_copy(accum.at[pl.ds(r1, REM_ROWS)], pbuf1.at[pl.ds(0, REM_ROWS)])
    pltpu.sync_copy(pbuf1.at[pl.ds(0, REM_ROWS)],
                    part_hbm.at[pl.ds(core * NUM_ENTITY + r1, REM_ROWS)])


def _combine_body(a_ref, b_ref, o_ref):
  o_ref[...] = a_ref[...] + b_ref[...]


_combine = pl.pallas_call(
    _combine_body,
    grid=(10,),
    in_specs=[
        pl.BlockSpec((NUM_ENTITY // 10, DIM), lambda i: (i, 0)),
        pl.BlockSpec((NUM_ENTITY // 10, DIM), lambda i: (i + 10, 0)),
    ],
    out_specs=pl.BlockSpec((NUM_ENTITY // 10, DIM), lambda i: (i, 0)),
    out_shape=jax.ShapeDtypeStruct((NUM_ENTITY, DIM), jnp.float32),
)


def _pack_bf16(table):
  n = table.shape[0]
  b = table.astype(jnp.bfloat16).reshape(n, DIM // 32, 2, L)
  b = b.transpose(0, 1, 3, 2)
  return lax.bitcast_convert_type(b, jnp.int32).reshape(n, DIM // 2)


def kernel(local_entity_emb, rel_emb, batch_heads, batch_rels, batch_tails,
           batch_ids):
  part = _sc_message_pass(_pack_bf16(local_entity_emb), _pack_bf16(rel_emb),
                          batch_heads, batch_rels, batch_ids, batch_tails)
  return _combine(part, part)


# speedup vs baseline: 1.6251x; 1.6251x over previous
import functools

import jax
import jax.numpy as jnp
from jax import lax
from jax.experimental import pallas as pl
from jax.experimental.pallas import tpu as pltpu
from jax.experimental.pallas import tpu_sc as plsc

NUM_ENTITY = 10000
NUM_RELATION = 200
NUM_FACT = 320000
DIM = 128

NC = 2
NS = 16
NW = NC * NS
L = 16

FACTS_PER_W = NUM_FACT // NW
BLK = 80
CHUNK = 2000
BLKS_PER_CHUNK = CHUNK // BLK
PAIRS = (BLKS_PER_CHUNK - 1) // 2
CHUNKS = FACTS_PER_W // CHUNK
ROWS_PER_TILE = 624
REM_ROWS = NUM_ENTITY - NS * ROWS_PER_TILE

_mesh = plsc.VectorSubcoreMesh(
    core_axis_name="c", subcore_axis_name="s", num_cores=NC, num_subcores=NS)


@functools.partial(
    pl.kernel,
    out_type=jax.ShapeDtypeStruct((NC * NUM_ENTITY, DIM), jnp.float32),
    mesh=_mesh,
    scratch_types=dict(
        hd_st=pltpu.VMEM((CHUNK,), jnp.int32),
        rl_st=pltpu.VMEM((CHUNK,), jnp.int32),
        bi_st=pltpu.VMEM((CHUNK,), jnp.int32),
        tl_st=pltpu.VMEM((CHUNK,), jnp.int32),
        ridx0=pltpu.VMEM((BLK,), jnp.int32),
        ridx1=pltpu.VMEM((BLK,), jnp.int32),
        tidx0=pltpu.VMEM((BLK,), jnp.int32),
        tidx1=pltpu.VMEM((BLK,), jnp.int32),
        hbuf0=pltpu.VMEM((BLK, DIM), jnp.float32),
        hbuf1=pltpu.VMEM((BLK, DIM), jnp.float32),
        pbuf0=pltpu.VMEM((BLK, DIM), jnp.float32),
        pbuf1=pltpu.VMEM((BLK, DIM), jnp.float32),
        accum=pltpu.VMEM_SHARED((NUM_ENTITY, DIM), jnp.float32),
        sem_st=pltpu.SemaphoreType.DMA,
        sem_h0=pltpu.SemaphoreType.DMA,
        sem_h1=pltpu.SemaphoreType.DMA,
        sem_r0=pltpu.SemaphoreType.DMA,
        sem_r1=pltpu.SemaphoreType.DMA,
    ),
    compiler_params=pltpu.CompilerParams(use_tc_tiling_on_sc=False),
)
def _sc_message_pass(entity_hbm, rel_hbm, heads_hbm, rels_hbm, ids_hbm,
                     tails_hbm, part_hbm, hd_st, rl_st, bi_st, tl_st, ridx0,
                     ridx1, tidx0, tidx1, hbuf0, hbuf1, pbuf0,
                     pbuf1, accum, sem_st, sem_h0, sem_h1, sem_r0, sem_r1):
  core = lax.axis_index("c")
  sid = lax.axis_index("s")
  w = core * NS + sid

  zero = jnp.zeros((L,), jnp.float32)

  def _zrow(r, _):
    for j in range(DIM // L):
      pbuf0[r, pl.ds(j * L, L)] = zero
    return 0
  lax.fori_loop(0, BLK, _zrow, 0)
  for k in range(7):
    pltpu.sync_copy(pbuf0,
                    accum.at[pl.ds(sid * ROWS_PER_TILE + k * BLK, BLK)])
  pltpu.sync_copy(pbuf0.at[pl.ds(0, 64)],
                  accum.at[pl.ds(sid * ROWS_PER_TILE + 7 * BLK, 64)])
  @pl.when(sid == NS - 1)
  def _zero_tail():
    pltpu.sync_copy(pbuf0.at[pl.ds(0, REM_ROWS)],
                    accum.at[pl.ds(NS * ROWS_PER_TILE, REM_ROWS)])
  plsc.subcore_barrier()

  bufs = (
      (ridx0, tidx0, hbuf0, pbuf0, sem_h0, sem_r0, pbuf0),
      (ridx1, tidx1, hbuf1, pbuf1, sem_h1, sem_r1, pbuf1),
  )

  def _idx(off, p):
    ridx, tidx = bufs[p][0], bufs[p][1]
    for j in range(BLK // L):
      s = pl.ds(j * L, L)
      src = pl.ds(off + j * L, L)
      ridx[s] = rl_st[src] + bi_st[src] * NUM_RELATION
      tidx[s] = tl_st[src]

  def _issue_gathers(off, p):
    ridx, _, hbuf, rbuf, sem_h, sem_r = bufs[p][:6]
    pltpu.async_copy(entity_hbm.at[hd_st.at[pl.ds(off, BLK)]], hbuf, sem_h)
    pltpu.async_copy(rel_hbm.at[ridx], rbuf, sem_r)

  def _wait_gathers(off, p):
    ridx, _, hbuf, rbuf, sem_h, sem_r = bufs[p][:6]
    pltpu.make_async_copy(entity_hbm.at[hd_st.at[pl.ds(off, BLK)]], hbuf,
                          sem_h).wait()
    pltpu.make_async_copy(rel_hbm.at[ridx], rbuf, sem_r).wait()

  hi_mask = jnp.full((L,), -65536, jnp.int32)
  sixteen = jnp.full((L,), 16, jnp.int32)

  def _mul(p):
    hbuf, rbuf = bufs[p][2], bufs[p][3]
    def _mrow(r, _):
      for j in range(DIM // L):
        s = pl.ds(j * L, L)
        rbuf[r, s] = hbuf[r, s] * rbuf[r, s]
      return 0
    lax.fori_loop(0, BLK, _mrow, 0)

  def _scatter(p):
    tidx, pbuf = bufs[p][1], bufs[p][6]
    pltpu.sync_copy(pbuf, accum.at[tidx], add=True)

  def _chunk(c, _):
    base = w * FACTS_PER_W + c * CHUNK
    cps = [
        pltpu.async_copy(heads_hbm.at[pl.ds(base, CHUNK)], hd_st, sem_st),
        pltpu.async_copy(rels_hbm.at[pl.ds(base, CHUNK)], rl_st, sem_st),
        pltpu.async_copy(ids_hbm.at[pl.ds(base, CHUNK)], bi_st, sem_st),
        pltpu.async_copy(tails_hbm.at[pl.ds(base, CHUNK)], tl_st, sem_st),
    ]
    for cp in cps:
      cp.wait()

    _idx(0, 0)
    _issue_gathers(0, 0)

    def _pair(i, _):
      b1 = 2 * i + 1
      _idx(b1 * BLK, 1)
      _issue_gathers(b1 * BLK, 1)
      _wait_gathers((b1 - 1) * BLK, 0)
      _mul(0)
      _scatter(0)

      b2 = 2 * i + 2
      _idx(b2 * BLK, 0)
      _issue_gathers(b2 * BLK, 0)
      _wait_gathers((b2 - 1) * BLK, 1)
      _mul(1)
      _scatter(1)
      return 0

    lax.fori_loop(0, PAIRS, _pair, 0)

    _wait_gathers((BLKS_PER_CHUNK - 1) * BLK, 0)
    _mul(0)
    _scatter(0)
    return 0

  lax.fori_loop(0, CHUNKS, _chunk, 0)

  plsc.subcore_barrier()
  for k in range(7):
    r0 = sid * ROWS_PER_TILE + k * BLK
    pltpu.sync_copy(accum.at[pl.ds(r0, BLK)], pbuf0)
    pltpu.sync_copy(pbuf0, part_hbm.at[pl.ds(core * NUM_ENTITY + r0, BLK)])
  r0 = sid * ROWS_PER_TILE + 7 * BLK
  pltpu.sync_copy(accum.at[pl.ds(r0, 64)], pbuf0.at[pl.ds(0, 64)])
  pltpu.sync_copy(pbuf0.at[pl.ds(0, 64)],
                  part_hbm.at[pl.ds(core * NUM_ENTITY + r0, 64)])
  @pl.when(sid == NS - 1)
  def _drain_tail():
    r1 = NS * ROWS_PER_TILE
    pltpu.sync_copy(accum.at[pl.ds(r1, REM_ROWS)], pbuf1.at[pl.ds(0, REM_ROWS)])
    pltpu.sync_copy(pbuf1.at[pl.ds(0, REM_ROWS)],
                    part_hbm.at[pl.ds(core * NUM_ENTITY + r1, REM_ROWS)])


def _combine_body(a_ref, b_ref, o_ref):
  o_ref[...] = a_ref[...] + b_ref[...]


_combine = pl.pallas_call(
    _combine_body,
    grid=(10,),
    in_specs=[
        pl.BlockSpec((NUM_ENTITY // 10, DIM), lambda i: (i, 0)),
        pl.BlockSpec((NUM_ENTITY // 10, DIM), lambda i: (i + 10, 0)),
    ],
    out_specs=pl.BlockSpec((NUM_ENTITY // 10, DIM), lambda i: (i, 0)),
    out_shape=jax.ShapeDtypeStruct((NUM_ENTITY, DIM), jnp.float32),
)


def _pack_bf16(table):
  n = table.shape[0]
  b = table.astype(jnp.bfloat16).reshape(n, DIM // 32, 2, L)
  b = b.transpose(0, 1, 3, 2)
  return lax.bitcast_convert_type(b, jnp.int32).reshape(n, DIM // 2)


def kernel(local_entity_emb, rel_emb, batch_heads, batch_rels, batch_tails,
           batch_ids):
  part = _sc_message_pass(local_entity_emb, rel_emb,
                          batch_heads, batch_rels, batch_ids, batch_tails)
  return _combine(part, part)
